# Initial kernel scaffold; baseline (speedup 1.0000x reference)
import jax
import jax.numpy as jnp
from jax.experimental import pallas as pl


def kernel(x, edge_index, W1a, b1a, W1b, b1b, W2a, b2a, W2b, b2b, Wl, bl):
    raise NotImplementedError("write your pallas kernel here")


# speedup vs baseline: 5.0137x; 5.0137x over previous
import functools

import jax
import jax.numpy as jnp
from jax import lax
from jax.experimental import pallas as pl
from jax.experimental.pallas import tpu as pltpu
from jax.experimental.pallas import tpu_sc as plsc

N = 10000
E = 320000
D = 128

NC = 2
NS = 16
NW = NC * NS
EPW = E // NW
C = 80
CPW = EPW // C
ROWM = 624
TAIL = N - NS * ROWM

@functools.cache
def _get_sc_agg():
    mesh = plsc.VectorSubcoreMesh(
        core_axis_name="c", subcore_axis_name="s",
        num_cores=NC, num_subcores=NS)

    @functools.partial(
        pl.kernel,
        out_type=jax.ShapeDtypeStruct((NC, N, D), jnp.float32),
        mesh=mesh,
        scratch_types=[
            pltpu.VMEM((C,), jnp.int32),
            pltpu.VMEM((C,), jnp.int32),
            pltpu.VMEM((C, D), jnp.float32),
            pltpu.VMEM_SHARED((N, D), jnp.float32),
            pltpu.SemaphoreType.DMA,
        ],
    )
    def _sc_agg(x_hbm, src_hbm, dst_hbm, zeros_hbm, out_hbm,
                src_v, dst_v, rows_v, acc_sh, sem):
        cid = lax.axis_index("c")
        sid = lax.axis_index("s")
        wid = cid * NS + sid

        pltpu.sync_copy(zeros_hbm, acc_sh.at[pl.ds(sid * ROWM, ROWM)])

        @pl.when(sid == 0)
        def _zero_tail():
            pltpu.sync_copy(zeros_hbm.at[pl.ds(0, TAIL)],
                            acc_sh.at[pl.ds(NS * ROWM, TAIL)])

        plsc.subcore_barrier()

        def body(i, carry):
            base = wid * EPW + i * C
            pltpu.sync_copy(src_hbm.at[pl.ds(base, C)], src_v)
            pltpu.sync_copy(dst_hbm.at[pl.ds(base, C)], dst_v)
            pltpu.async_copy(x_hbm.at[src_v], rows_v, sem).wait()
            pltpu.sync_copy(rows_v, acc_sh.at[dst_v], add=True)
            return carry

        lax.fori_loop(0, CPW, body, 0)
        plsc.subcore_barrier()

        pltpu.sync_copy(acc_sh.at[pl.ds(sid * ROWM, ROWM)],
                        out_hbm.at[cid, pl.ds(sid * ROWM, ROWM)])

        @pl.when(sid == 0)
        def _write_tail():
            pltpu.sync_copy(acc_sh.at[pl.ds(NS * ROWM, TAIL)],
                            out_hbm.at[cid, pl.ds(NS * ROWM, TAIL)])

    return _sc_agg


BN = 1000


def _mlp1_body(x_ref, p0_ref, p1_ref, wa_ref, ba_ref, wb_ref, bb_ref, o_ref):
    h = x_ref[...] + p0_ref[...] + p1_ref[...]
    h = jnp.dot(h, wa_ref[...], preferred_element_type=jnp.float32) + ba_ref[...]
    h = jnp.maximum(h, 0.0)
    h = jnp.dot(h, wb_ref[...], preferred_element_type=jnp.float32) + bb_ref[...]
    o_ref[...] = jnp.maximum(h, 0.0)


def _mlp2_body(x_ref, p0_ref, p1_ref, wa_ref, ba_ref, wb_ref, bb_ref,
               wl_ref, bl_ref, o_ref):
    h = x_ref[...] + p0_ref[...] + p1_ref[...]
    h = jnp.dot(h, wa_ref[...], preferred_element_type=jnp.float32) + ba_ref[...]
    h = jnp.maximum(h, 0.0)
    h = jnp.dot(h, wb_ref[...], preferred_element_type=jnp.float32) + bb_ref[...]
    h = jnp.maximum(h, 0.0)
    o_ref[...] = jnp.dot(h, wl_ref[...], preferred_element_type=jnp.float32) + bl_ref[...]


def _row_block(bn, d):
    return pl.BlockSpec((bn, d), lambda i: (i, 0))


def _full_block(shape):
    return pl.BlockSpec(shape, lambda i: tuple(0 for _ in shape))


def _mlp1(x, p0, p1, wa, ba, wb, bb):
    return pl.pallas_call(
        _mlp1_body,
        out_shape=jax.ShapeDtypeStruct((N, D), jnp.float32),
        grid=(N // BN,),
        in_specs=[
            _row_block(BN, D), _row_block(BN, D), _row_block(BN, D),
            _full_block((D, D)), _full_block((1, D)),
            _full_block((D, D)), _full_block((1, D)),
        ],
        out_specs=_row_block(BN, D),
    )(x, p0, p1, wa, ba.reshape(1, D), wb, bb.reshape(1, D))


def _mlp2(x, p0, p1, wa, ba, wb, bb, wl, bl):
    return pl.pallas_call(
        _mlp2_body,
        out_shape=jax.ShapeDtypeStruct((N, D), jnp.float32),
        grid=(N // BN,),
        in_specs=[
            _row_block(BN, D), _row_block(BN, D), _row_block(BN, D),
            _full_block((D, D)), _full_block((1, D)),
            _full_block((D, D)), _full_block((1, D)),
            _full_block((D, D)), _full_block((1, D)),
        ],
        out_specs=_row_block(BN, D),
    )(x, p0, p1, wa, ba.reshape(1, D), wb, bb.reshape(1, D),
      wl, bl.reshape(1, D))


def kernel(x, edge_index, W1a, b1a, W1b, b1b, W2a, b2a, W2b, b2b, Wl, bl):
    src = edge_index[0]
    dst = edge_index[1]
    zeros = jnp.zeros((ROWM, D), jnp.float32)

    sc_agg = _get_sc_agg()
    p = sc_agg(x, src, dst, zeros)
    h1 = _mlp1(x, p[0], p[1], W1a, b1a, W1b, b1b)
    q = sc_agg(h1, src, dst, zeros)
    out = _mlp2(h1, q[0], q[1], W2a, b2a, W2b, b2b, Wl, bl)
    return out


# speedup vs baseline: 11.1731x; 2.2285x over previous
import functools

import jax
import jax.numpy as jnp
from jax import lax
from jax.experimental import pallas as pl
from jax.experimental.pallas import tpu as pltpu
from jax.experimental.pallas import tpu_sc as plsc

N = 10000
E = 320000
D = 128

NC = 2
NS = 16
NW = NC * NS
EPW = E // NW
C = 80
CPW = EPW // C
ROWM = 624
TAIL = N - NS * ROWM
NBUF = 4
LAG = 2

@functools.cache
def _get_sc_agg():
    mesh = plsc.VectorSubcoreMesh(
        core_axis_name="c", subcore_axis_name="s",
        num_cores=NC, num_subcores=NS)

    @functools.partial(
        pl.kernel,
        out_type=jax.ShapeDtypeStruct((NC, N, D), jnp.float32),
        mesh=mesh,
        scratch_types=(
            [pltpu.VMEM((C,), jnp.int32)] * NBUF
            + [pltpu.VMEM((C,), jnp.int32)] * NBUF
            + [pltpu.VMEM((C, D), jnp.float32)] * NBUF
            + [pltpu.VMEM_SHARED((N, D), jnp.float32)]
            + [pltpu.SemaphoreType.DMA] * (3 * NBUF)
        ),
    )
    def _sc_agg(x_hbm, src_hbm, dst_hbm, zeros_hbm, out_hbm, *scr):
        src_v = scr[:NBUF]
        dst_v = scr[NBUF:2 * NBUF]
        rows_v = scr[2 * NBUF:3 * NBUF]
        acc_sh = scr[3 * NBUF]
        sem_is = scr[3 * NBUF + 1:3 * NBUF + 1 + NBUF]
        sem_id = scr[3 * NBUF + 1 + NBUF:3 * NBUF + 1 + 2 * NBUF]
        sem_g = scr[3 * NBUF + 1 + 2 * NBUF:]
        cid = lax.axis_index("c")
        sid = lax.axis_index("s")
        wid = cid * NS + sid

        pltpu.sync_copy(zeros_hbm, acc_sh.at[pl.ds(sid * ROWM, ROWM)])

        @pl.when(sid == 0)
        def _zero_tail():
            pltpu.sync_copy(zeros_hbm.at[pl.ds(0, TAIL)],
                            acc_sh.at[pl.ds(NS * ROWM, TAIL)])

        plsc.subcore_barrier()

        def fire_idx(j, b):
            pltpu.async_copy(src_hbm.at[wid, j], src_v[b], sem_is[b])
            pltpu.async_copy(dst_hbm.at[wid, j], dst_v[b], sem_id[b])

        def wait_idx(b):
            pltpu.make_async_copy(src_hbm.at[wid, 0], src_v[b], sem_is[b]).wait()
            pltpu.make_async_copy(dst_hbm.at[wid, 0], dst_v[b], sem_id[b]).wait()

        def fire_gather(b):
            pltpu.async_copy(x_hbm.at[src_v[b]], rows_v[b], sem_g[b])

        def wait_gather(b):
            pltpu.make_async_copy(x_hbm.at[src_v[0]], rows_v[b], sem_g[b]).wait()

        def sync_scatter(b):
            pltpu.sync_copy(rows_v[b], acc_sh.at[dst_v[b]], add=True)

        for b in range(NBUF):
            fire_idx(b, b)
        for t in range(LAG):
            wait_idx(t)
            fire_gather(t)

        def outer(g, carry):
            for b in range(NBUF):
                i = g * NBUF + b
                wait_gather(b)
                sync_scatter(b)

                @pl.when(i + NBUF < CPW)
                def _prefetch_idx():
                    fire_idx(i + NBUF, b)

                @pl.when(i + LAG < CPW)
                def _prefetch_gather():
                    bg = (b + LAG) % NBUF
                    wait_idx(bg)
                    fire_gather(bg)
            return carry

        lax.fori_loop(0, CPW // NBUF, outer, 0)
        for i in range((CPW // NBUF) * NBUF, CPW):
            b = i % NBUF
            wait_gather(b)
            sync_scatter(b)
        plsc.subcore_barrier()

        pltpu.sync_copy(acc_sh.at[pl.ds(sid * ROWM, ROWM)],
                        out_hbm.at[cid, pl.ds(sid * ROWM, ROWM)])

        @pl.when(sid == 0)
        def _write_tail():
            pltpu.sync_copy(acc_sh.at[pl.ds(NS * ROWM, TAIL)],
                            out_hbm.at[cid, pl.ds(NS * ROWM, TAIL)])

    return _sc_agg


BN = 1000


def _mlp1_body(x_ref, p0_ref, p1_ref, wa_ref, ba_ref, wb_ref, bb_ref, o_ref):
    h = x_ref[...] + p0_ref[...] + p1_ref[...]
    h = jnp.dot(h, wa_ref[...], preferred_element_type=jnp.float32) + ba_ref[...]
    h = jnp.maximum(h, 0.0)
    h = jnp.dot(h, wb_ref[...], preferred_element_type=jnp.float32) + bb_ref[...]
    o_ref[...] = jnp.maximum(h, 0.0)


def _mlp2_body(x_ref, p0_ref, p1_ref, wa_ref, ba_ref, wb_ref, bb_ref,
               wl_ref, bl_ref, o_ref):
    h = x_ref[...] + p0_ref[...] + p1_ref[...]
    h = jnp.dot(h, wa_ref[...], preferred_element_type=jnp.float32) + ba_ref[...]
    h = jnp.maximum(h, 0.0)
    h = jnp.dot(h, wb_ref[...], preferred_element_type=jnp.float32) + bb_ref[...]
    h = jnp.maximum(h, 0.0)
    o_ref[...] = jnp.dot(h, wl_ref[...], preferred_element_type=jnp.float32) + bl_ref[...]


def _row_block(bn, d):
    return pl.BlockSpec((bn, d), lambda i: (i, 0))


def _full_block(shape):
    return pl.BlockSpec(shape, lambda i: tuple(0 for _ in shape))


def _mlp1(x, p0, p1, wa, ba, wb, bb):
    return pl.pallas_call(
        _mlp1_body,
        out_shape=jax.ShapeDtypeStruct((N, D), jnp.float32),
        grid=(N // BN,),
        in_specs=[
            _row_block(BN, D), _row_block(BN, D), _row_block(BN, D),
            _full_block((D, D)), _full_block((1, D)),
            _full_block((D, D)), _full_block((1, D)),
        ],
        out_specs=_row_block(BN, D),
    )(x, p0, p1, wa, ba.reshape(1, D), wb, bb.reshape(1, D))


def _mlp2(x, p0, p1, wa, ba, wb, bb, wl, bl):
    return pl.pallas_call(
        _mlp2_body,
        out_shape=jax.ShapeDtypeStruct((N, D), jnp.float32),
        grid=(N // BN,),
        in_specs=[
            _row_block(BN, D), _row_block(BN, D), _row_block(BN, D),
            _full_block((D, D)), _full_block((1, D)),
            _full_block((D, D)), _full_block((1, D)),
            _full_block((D, D)), _full_block((1, D)),
        ],
        out_specs=_row_block(BN, D),
    )(x, p0, p1, wa, ba.reshape(1, D), wb, bb.reshape(1, D),
      wl, bl.reshape(1, D))


def kernel(x, edge_index, W1a, b1a, W1b, b1b, W2a, b2a, W2b, b2b, Wl, bl):
    src = edge_index[0].reshape(NW, CPW, C)
    dst = edge_index[1].reshape(NW, CPW, C)
    zeros = jnp.zeros((ROWM, D), jnp.float32)

    sc_agg = _get_sc_agg()
    p = sc_agg(x, src, dst, zeros)
    h1 = _mlp1(x, p[0], p[1], W1a, b1a, W1b, b1b)
    q = sc_agg(h1, src, dst, zeros)
    out = _mlp2(h1, q[0], q[1], W2a, b2a, W2b, b2b, Wl, bl)
    return out


# speedup vs baseline: 11.3536x; 1.0162x over previous
import functools

import jax
import jax.numpy as jnp
from jax import lax
from jax.experimental import pallas as pl
from jax.experimental.pallas import tpu as pltpu
from jax.experimental.pallas import tpu_sc as plsc

N = 10000
E = 320000
D = 128

NC = 2
NS = 16
NW = NC * NS
EPW = E // NW
C = 80
CPW = EPW // C
ROWM = 624
TAIL = N - NS * ROWM
NBUF = 4
LAG = 2
NI = 2 * NBUF

@functools.cache
def _get_sc_agg():
    mesh = plsc.VectorSubcoreMesh(
        core_axis_name="c", subcore_axis_name="s",
        num_cores=NC, num_subcores=NS)

    @functools.partial(
        pl.kernel,
        out_type=jax.ShapeDtypeStruct((NC, N, D), jnp.float32),
        mesh=mesh,
        scratch_types=(
            [pltpu.VMEM((C,), jnp.int32)] * NI
            + [pltpu.VMEM((C,), jnp.int32)] * NI
            + [pltpu.VMEM((C, D), jnp.float32)] * NBUF
            + [pltpu.VMEM_SHARED((N, D), jnp.float32)]
            + [pltpu.SemaphoreType.DMA] * (2 * NI + 2 * NBUF)
        ),
    )
    def _sc_agg(x_hbm, src_hbm, dst_hbm, zeros_hbm, out_hbm, *scr):
        src_v = scr[:NI]
        dst_v = scr[NI:2 * NI]
        rows_v = scr[2 * NI:2 * NI + NBUF]
        acc_sh = scr[2 * NI + NBUF]
        s0 = 2 * NI + NBUF + 1
        sem_is = scr[s0:s0 + NI]
        sem_id = scr[s0 + NI:s0 + 2 * NI]
        sem_g = scr[s0 + 2 * NI:s0 + 2 * NI + NBUF]
        sem_s = scr[s0 + 2 * NI + NBUF:]
        cid = lax.axis_index("c")
        sid = lax.axis_index("s")
        wid = cid * NS + sid

        pltpu.sync_copy(zeros_hbm, acc_sh.at[pl.ds(sid * ROWM, ROWM)])

        @pl.when(sid == 0)
        def _zero_tail():
            pltpu.sync_copy(zeros_hbm.at[pl.ds(0, TAIL)],
                            acc_sh.at[pl.ds(NS * ROWM, TAIL)])

        plsc.subcore_barrier()

        def fire_idx(j, k):
            pltpu.async_copy(src_hbm.at[wid, j], src_v[k], sem_is[k])
            pltpu.async_copy(dst_hbm.at[wid, j], dst_v[k], sem_id[k])

        def wait_idx(k):
            pltpu.make_async_copy(src_hbm.at[wid, 0], src_v[k], sem_is[k]).wait()
            pltpu.make_async_copy(dst_hbm.at[wid, 0], dst_v[k], sem_id[k]).wait()

        def fire_gather(k, b):
            pltpu.async_copy(x_hbm.at[src_v[k]], rows_v[b], sem_g[b])

        def wait_gather(b):
            pltpu.make_async_copy(x_hbm.at[src_v[0]], rows_v[b], sem_g[b]).wait()

        def fire_scatter(k, b):
            pltpu.async_copy(rows_v[b], acc_sh.at[dst_v[k]], sem_s[b], add=True)

        def wait_scatter(b):
            pltpu.make_async_copy(
                rows_v[0], acc_sh.at[dst_v[0]], sem_s[b]).wait()

        def visit(i, b, k, when):
            wait_gather(b)
            fire_scatter(k, b)
            when(i >= LAG, lambda: wait_scatter((b + LAG) % NBUF))
            when(i + NI - LAG < CPW,
                 lambda: fire_idx(i + NI - LAG, (k - LAG) % NI))

            def _pref_gather():
                wait_idx((k + LAG) % NI)
                fire_gather((k + LAG) % NI, (b + LAG) % NBUF)

            when(i + LAG < CPW, _pref_gather)

        def when_traced(cond, fn):
            pl.when(cond)(fn)

        def when_static(cond, fn):
            if cond:
                fn()

        for j in range(NI - LAG):
            fire_idx(j, j)
        for t in range(LAG):
            wait_idx(t)
            fire_gather(t, t)

        def outer(g, carry):
            for k in range(NI):
                visit(g * NI + k, k % NBUF, k, when_traced)
            return carry

        lax.fori_loop(0, CPW // NI, outer, 0)
        for i in range((CPW // NI) * NI, CPW):
            visit(i, i % NBUF, i % NI, when_static)
        for t in range(LAG):
            wait_scatter((CPW - LAG + t) % NBUF)
        plsc.subcore_barrier()

        pltpu.sync_copy(acc_sh.at[pl.ds(sid * ROWM, ROWM)],
                        out_hbm.at[cid, pl.ds(sid * ROWM, ROWM)])

        @pl.when(sid == 0)
        def _write_tail():
            pltpu.sync_copy(acc_sh.at[pl.ds(NS * ROWM, TAIL)],
                            out_hbm.at[cid, pl.ds(NS * ROWM, TAIL)])

    return _sc_agg


BN = 1000


def _mlp1_body(x_ref, p0_ref, p1_ref, wa_ref, ba_ref, wb_ref, bb_ref, o_ref):
    h = x_ref[...] + p0_ref[...] + p1_ref[...]
    h = jnp.dot(h, wa_ref[...], preferred_element_type=jnp.float32) + ba_ref[...]
    h = jnp.maximum(h, 0.0)
    h = jnp.dot(h, wb_ref[...], preferred_element_type=jnp.float32) + bb_ref[...]
    o_ref[...] = jnp.maximum(h, 0.0)


def _mlp2_body(x_ref, p0_ref, p1_ref, wa_ref, ba_ref, wb_ref, bb_ref,
               wl_ref, bl_ref, o_ref):
    h = x_ref[...] + p0_ref[...] + p1_ref[...]
    h = jnp.dot(h, wa_ref[...], preferred_element_type=jnp.float32) + ba_ref[...]
    h = jnp.maximum(h, 0.0)
    h = jnp.dot(h, wb_ref[...], preferred_element_type=jnp.float32) + bb_ref[...]
    h = jnp.maximum(h, 0.0)
    o_ref[...] = jnp.dot(h, wl_ref[...], preferred_element_type=jnp.float32) + bl_ref[...]


def _row_block(bn, d):
    return pl.BlockSpec((bn, d), lambda i: (i, 0))


def _full_block(shape):
    return pl.BlockSpec(shape, lambda i: tuple(0 for _ in shape))


def _mlp1(x, p0, p1, wa, ba, wb, bb):
    return pl.pallas_call(
        _mlp1_body,
        out_shape=jax.ShapeDtypeStruct((N, D), jnp.float32),
        grid=(N // BN,),
        in_specs=[
            _row_block(BN, D), _row_block(BN, D), _row_block(BN, D),
            _full_block((D, D)), _full_block((1, D)),
            _full_block((D, D)), _full_block((1, D)),
        ],
        out_specs=_row_block(BN, D),
    )(x, p0, p1, wa, ba.reshape(1, D), wb, bb.reshape(1, D))


def _mlp2(x, p0, p1, wa, ba, wb, bb, wl, bl):
    return pl.pallas_call(
        _mlp2_body,
        out_shape=jax.ShapeDtypeStruct((N, D), jnp.float32),
        grid=(N // BN,),
        in_specs=[
            _row_block(BN, D), _row_block(BN, D), _row_block(BN, D),
            _full_block((D, D)), _full_block((1, D)),
            _full_block((D, D)), _full_block((1, D)),
            _full_block((D, D)), _full_block((1, D)),
        ],
        out_specs=_row_block(BN, D),
    )(x, p0, p1, wa, ba.reshape(1, D), wb, bb.reshape(1, D),
      wl, bl.reshape(1, D))


def kernel(x, edge_index, W1a, b1a, W1b, b1b, W2a, b2a, W2b, b2b, Wl, bl):
    src = edge_index[0].reshape(NW, CPW, C)
    dst = edge_index[1].reshape(NW, CPW, C)
    zeros = jnp.zeros((ROWM, D), jnp.float32)

    sc_agg = _get_sc_agg()
    p = sc_agg(x, src, dst, zeros)
    h1 = _mlp1(x, p[0], p[1], W1a, b1a, W1b, b1b)
    q = sc_agg(h1, src, dst, zeros)
    out = _mlp2(h1, q[0], q[1], W2a, b2a, W2b, b2b, Wl, bl)
    return out


# speedup vs baseline: 11.6468x; 1.0258x over previous
import functools

import jax
import jax.numpy as jnp
from jax import lax
from jax.experimental import pallas as pl
from jax.experimental.pallas import tpu as pltpu
from jax.experimental.pallas import tpu_sc as plsc

N = 10000
E = 320000
D = 128

NC = 2
NS = 16
NW = NC * NS
EPW = E // NW
C = 80
CPW = EPW // C
ROWM = 624
TAIL = N - NS * ROWM
NBUF = 4
LAG = 2
NI = 2 * NBUF

@functools.cache
def _get_sc_agg():
    mesh = plsc.VectorSubcoreMesh(
        core_axis_name="c", subcore_axis_name="s",
        num_cores=NC, num_subcores=NS)

    @functools.partial(
        pl.kernel,
        out_type=jax.ShapeDtypeStruct((NC, N, D), jnp.float32),
        mesh=mesh,
        scratch_types=(
            [pltpu.VMEM((2, C), jnp.int32)] * NI
            + [pltpu.VMEM((C, D), jnp.float32)] * NBUF
            + [pltpu.VMEM_SHARED((N, D), jnp.float32)]
            + [pltpu.SemaphoreType.DMA] * (NI + 2 * NBUF)
        ),
    )
    def _sc_agg(x_hbm, eidx_hbm, zeros_hbm, out_hbm, *scr):
        ib = scr[:NI]
        rows_v = scr[NI:NI + NBUF]
        acc_sh = scr[NI + NBUF]
        s0 = NI + NBUF + 1
        sem_i = scr[s0:s0 + NI]
        sem_g = scr[s0 + NI:s0 + NI + NBUF]
        sem_s = scr[s0 + NI + NBUF:]
        cid = lax.axis_index("c")
        sid = lax.axis_index("s")
        wid = cid * NS + sid

        pltpu.sync_copy(zeros_hbm, acc_sh.at[pl.ds(sid * ROWM, ROWM)])

        @pl.when(sid == 0)
        def _zero_tail():
            pltpu.sync_copy(zeros_hbm.at[pl.ds(0, TAIL)],
                            acc_sh.at[pl.ds(NS * ROWM, TAIL)])

        plsc.subcore_barrier()

        def fire_idx(j, k):
            pltpu.async_copy(eidx_hbm.at[wid, j], ib[k], sem_i[k])

        def wait_idx(k):
            pltpu.make_async_copy(eidx_hbm.at[wid, 0], ib[k], sem_i[k]).wait()

        def fire_gather(k, b):
            pltpu.async_copy(x_hbm.at[ib[k].at[0]], rows_v[b], sem_g[b])

        def wait_gather(b):
            pltpu.make_async_copy(
                x_hbm.at[ib[0].at[0]], rows_v[b], sem_g[b]).wait()

        def fire_scatter(k, b):
            pltpu.async_copy(rows_v[b], acc_sh.at[ib[k].at[1]],
                             sem_s[b], add=True)

        def wait_scatter(b):
            pltpu.make_async_copy(
                rows_v[0], acc_sh.at[ib[0].at[1]], sem_s[b]).wait()

        def visit(i, b, k, when):
            wait_gather(b)
            fire_scatter(k, b)
            when(i >= LAG, lambda: wait_scatter((b + LAG) % NBUF))
            when(i + NI - LAG < CPW,
                 lambda: fire_idx(i + NI - LAG, (k - LAG) % NI))

            def _pref_gather():
                wait_idx((k + LAG) % NI)
                fire_gather((k + LAG) % NI, (b + LAG) % NBUF)

            when(i + LAG < CPW, _pref_gather)

        def when_traced(cond, fn):
            pl.when(cond)(fn)

        def when_static(cond, fn):
            if cond:
                fn()

        for j in range(NI - LAG):
            fire_idx(j, j)
        for t in range(LAG):
            wait_idx(t)
            fire_gather(t, t)

        def outer(g, carry):
            for k in range(NI):
                visit(g * NI + k, k % NBUF, k, when_traced)
            return carry

        lax.fori_loop(0, CPW // NI, outer, 0)
        for i in range((CPW // NI) * NI, CPW):
            visit(i, i % NBUF, i % NI, when_static)
        for t in range(LAG):
            wait_scatter((CPW - LAG + t) % NBUF)
        plsc.subcore_barrier()

        pltpu.sync_copy(acc_sh.at[pl.ds(sid * ROWM, ROWM)],
                        out_hbm.at[cid, pl.ds(sid * ROWM, ROWM)])

        @pl.when(sid == 0)
        def _write_tail():
            pltpu.sync_copy(acc_sh.at[pl.ds(NS * ROWM, TAIL)],
                            out_hbm.at[cid, pl.ds(NS * ROWM, TAIL)])

    return _sc_agg


BN = 1000


def _mlp1_body(x_ref, p0_ref, p1_ref, wa_ref, ba_ref, wb_ref, bb_ref, o_ref):
    h = x_ref[...] + p0_ref[...] + p1_ref[...]
    h = jnp.dot(h, wa_ref[...], preferred_element_type=jnp.float32) + ba_ref[...]
    h = jnp.maximum(h, 0.0)
    h = jnp.dot(h, wb_ref[...], preferred_element_type=jnp.float32) + bb_ref[...]
    o_ref[...] = jnp.maximum(h, 0.0)


def _mlp2_body(x_ref, p0_ref, p1_ref, wa_ref, ba_ref, wb_ref, bb_ref,
               wl_ref, bl_ref, o_ref):
    h = x_ref[...] + p0_ref[...] + p1_ref[...]
    h = jnp.dot(h, wa_ref[...], preferred_element_type=jnp.float32) + ba_ref[...]
    h = jnp.maximum(h, 0.0)
    h = jnp.dot(h, wb_ref[...], preferred_element_type=jnp.float32) + bb_ref[...]
    h = jnp.maximum(h, 0.0)
    o_ref[...] = jnp.dot(h, wl_ref[...], preferred_element_type=jnp.float32) + bl_ref[...]


def _row_block(bn, d):
    return pl.BlockSpec((bn, d), lambda i: (i, 0))


def _full_block(shape):
    return pl.BlockSpec(shape, lambda i: tuple(0 for _ in shape))


def _mlp1(x, p0, p1, wa, ba, wb, bb):
    return pl.pallas_call(
        _mlp1_body,
        out_shape=jax.ShapeDtypeStruct((N, D), jnp.float32),
        grid=(N // BN,),
        in_specs=[
            _row_block(BN, D), _row_block(BN, D), _row_block(BN, D),
            _full_block((D, D)), _full_block((1, D)),
            _full_block((D, D)), _full_block((1, D)),
        ],
        out_specs=_row_block(BN, D),
    )(x, p0, p1, wa, ba.reshape(1, D), wb, bb.reshape(1, D))


def _mlp2(x, p0, p1, wa, ba, wb, bb, wl, bl):
    return pl.pallas_call(
        _mlp2_body,
        out_shape=jax.ShapeDtypeStruct((N, D), jnp.float32),
        grid=(N // BN,),
        in_specs=[
            _row_block(BN, D), _row_block(BN, D), _row_block(BN, D),
            _full_block((D, D)), _full_block((1, D)),
            _full_block((D, D)), _full_block((1, D)),
            _full_block((D, D)), _full_block((1, D)),
        ],
        out_specs=_row_block(BN, D),
    )(x, p0, p1, wa, ba.reshape(1, D), wb, bb.reshape(1, D),
      wl, bl.reshape(1, D))


def kernel(x, edge_index, W1a, b1a, W1b, b1b, W2a, b2a, W2b, b2b, Wl, bl):
    eidx = jnp.transpose(edge_index.reshape(2, NW, CPW, C), (1, 2, 0, 3))
    zeros = jnp.zeros((ROWM, D), jnp.float32)

    sc_agg = _get_sc_agg()
    p = sc_agg(x, eidx, zeros)
    h1 = _mlp1(x, p[0], p[1], W1a, b1a, W1b, b1b)
    q = sc_agg(h1, eidx, zeros)
    out = _mlp2(h1, q[0], q[1], W2a, b2a, W2b, b2b, Wl, bl)
    return out


# speedup vs baseline: 12.4217x; 1.0665x over previous
import functools

import jax
import jax.numpy as jnp
from jax import lax
from jax.experimental import pallas as pl
from jax.experimental.pallas import tpu as pltpu
from jax.experimental.pallas import tpu_sc as plsc

N = 10000
E = 320000
D = 128

NC = 2
NS = 16
NW = NC * NS
EPW = E // NW
C = 80
CPW = EPW // C
ROWM = 624
TAIL = N - NS * ROWM
NBUF = 4
LAG = 2
NI = 2 * NBUF

@functools.cache
def _get_sc_agg():
    mesh = plsc.VectorSubcoreMesh(
        core_axis_name="c", subcore_axis_name="s",
        num_cores=NC, num_subcores=NS)

    @functools.partial(
        pl.kernel,
        out_type=jax.ShapeDtypeStruct((NC, N, D), jnp.float32),
        mesh=mesh,
        scratch_types=(
            [pltpu.VMEM((2, C), jnp.int32)] * NI
            + [pltpu.VMEM((C, D), jnp.float32)] * NBUF
            + [pltpu.VMEM_SHARED((N, D), jnp.float32)]
            + [pltpu.SemaphoreType.DMA] * (NI + 2 * NBUF)
        ),
    )
    def _sc_agg(x_hbm, eidx_hbm, zeros_hbm, out_hbm, *scr):
        ib = scr[:NI]
        rows_v = scr[NI:NI + NBUF]
        acc_sh = scr[NI + NBUF]
        s0 = NI + NBUF + 1
        sem_i = scr[s0:s0 + NI]
        sem_g = scr[s0 + NI:s0 + NI + NBUF]
        sem_s = scr[s0 + NI + NBUF:]
        cid = lax.axis_index("c")
        sid = lax.axis_index("s")
        wid = cid * NS + sid

        @pl.when(cid == 0)
        def _init_x():
            pltpu.sync_copy(x_hbm.at[pl.ds(sid * ROWM, ROWM)],
                            acc_sh.at[pl.ds(sid * ROWM, ROWM)])

            @pl.when(sid == 0)
            def _tail():
                pltpu.sync_copy(x_hbm.at[pl.ds(NS * ROWM, TAIL)],
                                acc_sh.at[pl.ds(NS * ROWM, TAIL)])

        @pl.when(cid == 1)
        def _init_zero():
            pltpu.sync_copy(zeros_hbm, acc_sh.at[pl.ds(sid * ROWM, ROWM)])

            @pl.when(sid == 0)
            def _tail():
                pltpu.sync_copy(zeros_hbm.at[pl.ds(0, TAIL)],
                                acc_sh.at[pl.ds(NS * ROWM, TAIL)])

        plsc.subcore_barrier()

        def fire_idx(j, k):
            pltpu.async_copy(eidx_hbm.at[wid, j], ib[k], sem_i[k])

        def wait_idx(k):
            pltpu.make_async_copy(eidx_hbm.at[wid, 0], ib[k], sem_i[k]).wait()

        def fire_gather(k, b):
            pltpu.async_copy(x_hbm.at[ib[k].at[0]], rows_v[b], sem_g[b])

        def wait_gather(b):
            pltpu.make_async_copy(
                x_hbm.at[ib[0].at[0]], rows_v[b], sem_g[b]).wait()

        def fire_scatter(k, b):
            pltpu.async_copy(rows_v[b], acc_sh.at[ib[k].at[1]],
                             sem_s[b], add=True)

        def wait_scatter(b):
            pltpu.make_async_copy(
                rows_v[0], acc_sh.at[ib[0].at[1]], sem_s[b]).wait()

        def visit(i, b, k, when):
            wait_gather(b)
            fire_scatter(k, b)
            when(i >= LAG, lambda: wait_scatter((b + LAG) % NBUF))
            when(i + NI - LAG < CPW,
                 lambda: fire_idx(i + NI - LAG, (k - LAG) % NI))

            def _pref_gather():
                wait_idx((k + LAG) % NI)
                fire_gather((k + LAG) % NI, (b + LAG) % NBUF)

            when(i + LAG < CPW, _pref_gather)

        def when_traced(cond, fn):
            pl.when(cond)(fn)

        def when_static(cond, fn):
            if cond:
                fn()

        for j in range(NI - LAG):
            fire_idx(j, j)
        for t in range(LAG):
            wait_idx(t)
            fire_gather(t, t)

        def outer(g, carry):
            for k in range(NI):
                visit(g * NI + k, k % NBUF, k, when_traced)
            return carry

        lax.fori_loop(0, CPW // NI, outer, 0)
        for i in range((CPW // NI) * NI, CPW):
            visit(i, i % NBUF, i % NI, when_static)
        for t in range(LAG):
            wait_scatter((CPW - LAG + t) % NBUF)
        plsc.subcore_barrier()

        pltpu.sync_copy(acc_sh.at[pl.ds(sid * ROWM, ROWM)],
                        out_hbm.at[cid, pl.ds(sid * ROWM, ROWM)])

        @pl.when(sid == 0)
        def _write_tail():
            pltpu.sync_copy(acc_sh.at[pl.ds(NS * ROWM, TAIL)],
                            out_hbm.at[cid, pl.ds(NS * ROWM, TAIL)])

    return _sc_agg


BN = 1000


def _mlp1_body(p0_ref, p1_ref, wa_ref, ba_ref, wb_ref, bb_ref, o_ref):
    h = p0_ref[0] + p1_ref[0]
    h = jnp.dot(h, wa_ref[...], preferred_element_type=jnp.float32) + ba_ref[...]
    h = jnp.maximum(h, 0.0)
    h = jnp.dot(h, wb_ref[...], preferred_element_type=jnp.float32) + bb_ref[...]
    o_ref[...] = jnp.maximum(h, 0.0)


def _mlp2_body(p0_ref, p1_ref, wa_ref, ba_ref, wb_ref, bb_ref,
               wl_ref, bl_ref, o_ref):
    h = p0_ref[0] + p1_ref[0]
    h = jnp.dot(h, wa_ref[...], preferred_element_type=jnp.float32) + ba_ref[...]
    h = jnp.maximum(h, 0.0)
    h = jnp.dot(h, wb_ref[...], preferred_element_type=jnp.float32) + bb_ref[...]
    h = jnp.maximum(h, 0.0)
    o_ref[...] = jnp.dot(h, wl_ref[...], preferred_element_type=jnp.float32) + bl_ref[...]


def _plane_block(c):
    return pl.BlockSpec((1, BN, D), lambda i, c=c: (c, i, 0))


def _row_block(bn, d):
    return pl.BlockSpec((bn, d), lambda i: (i, 0))


def _full_block(shape):
    return pl.BlockSpec(shape, lambda i: tuple(0 for _ in shape))


def _mlp1(p, wa, ba, wb, bb):
    return pl.pallas_call(
        _mlp1_body,
        out_shape=jax.ShapeDtypeStruct((N, D), jnp.float32),
        grid=(N // BN,),
        in_specs=[
            _plane_block(0), _plane_block(1),
            _full_block((D, D)), _full_block((1, D)),
            _full_block((D, D)), _full_block((1, D)),
        ],
        out_specs=_row_block(BN, D),
    )(p, p, wa, ba.reshape(1, D), wb, bb.reshape(1, D))


def _mlp2(p, wa, ba, wb, bb, wl, bl):
    return pl.pallas_call(
        _mlp2_body,
        out_shape=jax.ShapeDtypeStruct((N, D), jnp.float32),
        grid=(N // BN,),
        in_specs=[
            _plane_block(0), _plane_block(1),
            _full_block((D, D)), _full_block((1, D)),
            _full_block((D, D)), _full_block((1, D)),
            _full_block((D, D)), _full_block((1, D)),
        ],
        out_specs=_row_block(BN, D),
    )(p, p, wa, ba.reshape(1, D), wb, bb.reshape(1, D),
      wl, bl.reshape(1, D))


def kernel(x, edge_index, W1a, b1a, W1b, b1b, W2a, b2a, W2b, b2b, Wl, bl):
    eidx = jnp.transpose(edge_index.reshape(2, NW, CPW, C), (1, 2, 0, 3))
    zeros = jnp.zeros((ROWM, D), jnp.float32)

    sc_agg = _get_sc_agg()
    p = sc_agg(x, eidx, zeros)
    h1 = _mlp1(p, W1a, b1a, W1b, b1b)
    q = sc_agg(h1, eidx, zeros)
    out = _mlp2(q, W2a, b2a, W2b, b2b, Wl, bl)
    return out
